# baseline (speedup 1.0000x reference)
import functools

import jax
import jax.numpy as jnp
from jax import lax
from jax.experimental import pallas as pl
from jax.experimental.pallas import tpu as pltpu
from jax.experimental.pallas import tpu_sc as plsc

_NVOC = 1000000
_NUM_BINS = _NVOC + 1
_EMB_DIM = 32
_BATCH = 16384
_NW = 32
_BPW = _BATCH // _NW
_CHUNK = 128
_NCHUNK = _BPW // _CHUNK
_LANES = 16


def _hash_lanes(h, salt0, salt1):
    h = h * jnp.uint32(salt0) + jnp.uint32(salt1)
    h = h ^ (h >> jnp.uint32(16))
    h = h * jnp.uint32(0x45D9F3B)
    h = h ^ (h >> jnp.uint32(16))
    return (h % jnp.uint32(_NUM_BINS)).astype(jnp.int32)


def _body(x_hbm, t1_hbm, t2_hbm, out_hbm, x_v, idx1_v, idx2_v, r1_v, r2_v, sem):
    wid = lax.axis_index("s") * 2 + lax.axis_index("c")
    base = wid * _BPW
    pltpu.sync_copy(x_hbm.at[pl.ds(base, _BPW)], x_v)

    copies = []
    for j in range(_NCHUNK):
        def hash_step(k, _, j=j):
            xv = x_v[pl.ds(j * _CHUNK + k * _LANES, _LANES)].astype(jnp.uint32)
            idx1_v[j, pl.ds(k * _LANES, _LANES)] = _hash_lanes(xv, 6971, 7321)
            idx2_v[j, pl.ds(k * _LANES, _LANES)] = _hash_lanes(xv, 7723, 7507)
            return 0
        lax.fori_loop(0, _CHUNK // _LANES, hash_step, 0)
        copies.append(pltpu.async_copy(
            t1_hbm.at[idx1_v.at[j]], r1_v.at[pl.ds(j * _CHUNK, _CHUNK)], sem))
        copies.append(pltpu.async_copy(
            t2_hbm.at[idx2_v.at[j]], r2_v.at[pl.ds(j * _CHUNK, _CHUNK)], sem))
    for c in copies:
        c.wait()

    def mul_step(r, _):
        for h in range(_EMB_DIM // _LANES):
            sl = pl.ds(h * _LANES, _LANES)
            r1_v[r, sl] = r1_v[r, sl] * r2_v[r, sl]
        return 0
    lax.fori_loop(0, _BPW, mul_step, 0)

    pltpu.sync_copy(r1_v, out_hbm.at[pl.ds(base, _BPW)])


@jax.jit
def kernel(x, table1, table2):
    mesh = plsc.VectorSubcoreMesh(core_axis_name="c", subcore_axis_name="s")
    run = pl.kernel(
        _body,
        mesh=mesh,
        compiler_params=pltpu.CompilerParams(use_tc_tiling_on_sc=False),
        out_type=jax.ShapeDtypeStruct((_BATCH, _EMB_DIM), jnp.float32),
        scratch_types=[
            pltpu.VMEM((_BPW,), jnp.int32),
            pltpu.VMEM((_NCHUNK, _CHUNK), jnp.int32),
            pltpu.VMEM((_NCHUNK, _CHUNK), jnp.int32),
            pltpu.VMEM((_BPW, _EMB_DIM), jnp.float32),
            pltpu.VMEM((_BPW, _EMB_DIM), jnp.float32),
            pltpu.SemaphoreType.DMA,
        ],
    )
    return run(x.astype(jnp.int32), table1, table2)
